# baseline (speedup 1.0000x reference)
import functools

import jax
import jax.numpy as jnp
from jax.experimental import pallas as pl

P = 32768
A = 2048
ANCHOR_W = 4.0
ANCHOR_L = 4.0
ANCHOR_H = 15.0
NMS_IOU = 0.3
TOPK = 256

BA = 256
BP = 2048


def _score_body(cx_ref, cy_ref, px_ref, py_ref, pz_ref, sum_ref, cnt_ref):
    j = pl.program_id(1)

    @pl.when(j == 0)
    def _():
        sum_ref[...] = jnp.zeros_like(sum_ref)
        cnt_ref[...] = jnp.zeros_like(cnt_ref)

    cx = cx_ref[...]
    cy = cy_ref[...]
    px = px_ref[...]
    py = py_ref[...]
    pz = pz_ref[...]

    half_w = jnp.float32(ANCHOR_W) / 2
    half_l = jnp.float32(ANCHOR_L) / 2
    mask = ((px >= cx - half_w) & (px <= cx + half_w)
            & (py >= cy - half_l) & (py <= cy + half_l)
            & (pz >= 0) & (pz <= jnp.float32(ANCHOR_H)))
    r2 = (px - cx) ** 2 + (py - cy) ** 2
    denom = half_w ** 2 + jnp.float32(1e-6)
    weight = jnp.exp(-r2 / denom)
    contrib = jnp.where(mask, weight, jnp.float32(0.0))
    sum_ref[...] += contrib.sum(axis=1, keepdims=True)
    cnt_ref[...] += mask.astype(jnp.float32).sum(axis=1, keepdims=True)


def _nms_body(sum_c, cnt_c, sum_r, cnt_r, cx_c, cy_c, cx_r, cy_r,
              supp_ref, score_ref):
    s_i = sum_c[...] / (cnt_c[...] + 1.0)
    s_j = sum_r[...] / (cnt_r[...] + 1.0)
    score_ref[...] = s_i

    half_w = jnp.float32(ANCHOR_W) / 2
    half_l = jnp.float32(ANCHOR_L) / 2
    x1_i = cx_c[...] - half_w
    y1_i = cy_c[...] - half_l
    x2_i = cx_c[...] + half_w
    y2_i = cy_c[...] + half_l
    x1_j = cx_r[...] - half_w
    y1_j = cy_r[...] - half_l
    x2_j = cx_r[...] + half_w
    y2_j = cy_r[...] + half_l
    area_i = (x2_i - x1_i) * (y2_i - y1_i)
    area_j = (x2_j - x1_j) * (y2_j - y1_j)
    ix1 = jnp.maximum(x1_i, x1_j)
    iy1 = jnp.maximum(y1_i, y1_j)
    ix2 = jnp.minimum(x2_i, x2_j)
    iy2 = jnp.minimum(y2_i, y2_j)
    iw = jnp.clip(ix2 - ix1, 0.0, None)
    ih = jnp.clip(iy2 - iy1, 0.0, None)
    inter = iw * ih
    union = area_i + area_j - inter
    iou = inter / (union + jnp.float32(1e-9))

    i_blk = pl.program_id(0)
    idx_i = i_blk * BA + jax.lax.broadcasted_iota(jnp.int32, (BA, A), 0)
    idx_j = jax.lax.broadcasted_iota(jnp.int32, (BA, A), 1)
    higher = (s_j > s_i) | ((s_j == s_i) & (idx_j < idx_i))
    suppressed = jnp.any(higher & (iou > jnp.float32(NMS_IOU)), axis=1,
                         keepdims=True)
    supp_ref[...] = suppressed.astype(jnp.float32)


def _rank_body(score_c, supp_c, score_r, supp_r, rank_ref):
    neg_inf = jnp.float32(-jnp.inf)
    m_i = jnp.where(supp_c[...] > 0, neg_inf, score_c[...])
    m_j = jnp.where(supp_r[...] > 0, neg_inf, score_r[...])
    i_blk = pl.program_id(0)
    idx_i = i_blk * BA + jax.lax.broadcasted_iota(jnp.int32, (BA, A), 0)
    idx_j = jax.lax.broadcasted_iota(jnp.int32, (BA, A), 1)
    ahead = (m_j > m_i) | ((m_j == m_i) & (idx_j < idx_i))
    rank_ref[...] = ahead.astype(jnp.int32).sum(axis=1, keepdims=True)


def _select_body(rank_r, score_r, supp_r, cx_r, cy_r, boxes_ref, top_ref):
    neg_inf = jnp.float32(-jnp.inf)
    m_j = jnp.where(supp_r[...] > 0, neg_inf, score_r[...])
    k = jax.lax.broadcasted_iota(jnp.int32, (TOPK, A), 0)
    eq = rank_r[...] == k
    zero = jnp.float32(0.0)
    top_ref[...] = jnp.where(eq, m_j, zero).sum(axis=1, keepdims=True)
    bx = jnp.where(eq, cx_r[...], zero).sum(axis=1, keepdims=True)
    by = jnp.where(eq, cy_r[...], zero).sum(axis=1, keepdims=True)
    ones = jnp.ones((TOPK, 1), jnp.float32)
    boxes_ref[...] = jnp.concatenate(
        [bx, by, jnp.zeros((TOPK, 1), jnp.float32),
         ones * jnp.float32(ANCHOR_W), ones * jnp.float32(ANCHOR_L),
         ones * jnp.float32(ANCHOR_H)], axis=1)


def kernel(points, gt_boxes, local_maxima, plot_bounds, training):
    del gt_boxes, plot_bounds, training
    f32 = jnp.float32
    cx = local_maxima[:, 0:1].astype(f32)
    cy = local_maxima[:, 1:2].astype(f32)
    cx_r = cx.reshape(1, A)
    cy_r = cy.reshape(1, A)
    px = points[:, 0].reshape(1, P).astype(f32)
    py = points[:, 1].reshape(1, P).astype(f32)
    pz = points[:, 2].reshape(1, P).astype(f32)

    col = pl.BlockSpec((BA, 1), lambda i, j: (i, 0))
    row_pt = pl.BlockSpec((1, BP), lambda i, j: (0, j))
    s_sum, s_cnt = pl.pallas_call(
        _score_body,
        grid=(A // BA, P // BP),
        in_specs=[col, col, row_pt, row_pt, row_pt],
        out_specs=[col, col],
        out_shape=[jax.ShapeDtypeStruct((A, 1), f32),
                   jax.ShapeDtypeStruct((A, 1), f32)],
    )(cx, cy, px, py, pz)

    colb = pl.BlockSpec((BA, 1), lambda i: (i, 0))
    rowb = pl.BlockSpec((1, A), lambda i: (0, 0))
    sum_r = s_sum.reshape(1, A)
    cnt_r = s_cnt.reshape(1, A)
    supp, score = pl.pallas_call(
        _nms_body,
        grid=(A // BA,),
        in_specs=[colb, colb, rowb, rowb, colb, colb, rowb, rowb],
        out_specs=[colb, colb],
        out_shape=[jax.ShapeDtypeStruct((A, 1), f32),
                   jax.ShapeDtypeStruct((A, 1), f32)],
    )(s_sum, s_cnt, sum_r, cnt_r, cx, cy, cx_r, cy_r)

    score_r = score.reshape(1, A)
    supp_r = supp.reshape(1, A)
    rank = pl.pallas_call(
        _rank_body,
        grid=(A // BA,),
        in_specs=[colb, colb, rowb, rowb],
        out_specs=colb,
        out_shape=jax.ShapeDtypeStruct((A, 1), jnp.int32),
    )(score, supp, score_r, supp_r)

    rank_r = rank.reshape(1, A)
    full = pl.BlockSpec((1, A), lambda: (0, 0))
    boxes, top = pl.pallas_call(
        _select_body,
        in_specs=[full, full, full, full, full],
        out_specs=[pl.BlockSpec((TOPK, 6), lambda: (0, 0)),
                   pl.BlockSpec((TOPK, 1), lambda: (0, 0))],
        out_shape=[jax.ShapeDtypeStruct((TOPK, 6), f32),
                   jax.ShapeDtypeStruct((TOPK, 1), f32)],
    )(rank_r, score_r, supp_r, cx_r, cy_r)

    return boxes, top.reshape(TOPK)


# speedup vs baseline: 1.1599x; 1.1599x over previous
import functools

import jax
import jax.numpy as jnp
from jax import lax
from jax.experimental import pallas as pl
from jax.experimental.pallas import tpu as pltpu, tpu_sc as plsc

P = 32768
A = 2048
ANCHOR_W = 4.0
ANCHOR_L = 4.0
ANCHOR_H = 15.0
NMS_IOU = 0.3
TOPK = 256

NW = 32
PPT = P // NW
APT = A // NW
G = 25
NB = G * G
CAP = 128
OVSTART = NB * CAP
NROWS = OVSTART + P
STRIPE = NROWS // 16
NBPAD = 640

BA = 256

_mesh = plsc.VectorSubcoreMesh(core_axis_name="c", subcore_axis_name="s")
_sc_params = pltpu.CompilerParams(needs_layout_passes=False)

_i32 = jnp.int32
_f32 = jnp.float32


def _bin_ids(xv, yv):
    bx = jnp.clip((xv * 0.25).astype(_i32), 0, G - 1)
    by = jnp.clip((yv * 0.25).astype(_i32), 0, G - 1)
    return bx * G + by


def _gv(ref, i):
    return jnp.max(plsc.load_gather(ref, [jnp.full((16,), i, _i32)]))


@functools.partial(
    pl.kernel, mesh=_mesh, compiler_params=_sc_params,
    out_type=jax.ShapeDtypeStruct((NW * NBPAD,), _i32),
    scratch_types=[pltpu.VMEM((PPT,), _f32),
                   pltpu.VMEM((PPT,), _f32),
                   pltpu.VMEM((NBPAD,), _i32)],
)
def _sc_hist(px, py, hist_out, pxl, pyl, histl):
    cid = lax.axis_index("c")
    sid = lax.axis_index("s")
    wid = cid * 16 + sid
    base = pl.multiple_of(wid * PPT, PPT)
    pltpu.sync_copy(px.at[pl.ds(base, PPT)], pxl)
    pltpu.sync_copy(py.at[pl.ds(base, PPT)], pyl)

    def zero(c, carry):
        histl[pl.ds(pl.multiple_of(c * 16, 16), 16)] = jnp.zeros((16,), _i32)
        return carry

    lax.fori_loop(0, NBPAD // 16, zero, 0)

    def chunk(k, carry):
        o = pl.multiple_of(k * 16, 16)
        b = _bin_ids(pxl[pl.ds(o, 16)], pyl[pl.ds(o, 16)])
        cnt, last = plsc.scan_count(b)
        cur = plsc.load_gather(histl, [b])
        plsc.store_scatter(histl, [b], cur + cnt, mask=last)
        return carry

    lax.fori_loop(0, PPT // 16, chunk, 0)
    hbase = pl.multiple_of(wid * NBPAD, NBPAD)
    pltpu.sync_copy(histl, hist_out.at[pl.ds(hbase, NBPAD)])


@functools.partial(
    pl.kernel, mesh=_mesh, compiler_params=_sc_params,
    out_type=[jax.ShapeDtypeStruct((NROWS,), _f32),
              jax.ShapeDtypeStruct((NROWS,), _f32),
              jax.ShapeDtypeStruct((NROWS,), _f32),
              jax.ShapeDtypeStruct((NW * 16,), _i32)],
    scratch_types=[pltpu.VMEM((PPT,), _f32),
                   pltpu.VMEM((PPT,), _f32),
                   pltpu.VMEM((PPT,), _f32),
                   pltpu.VMEM((NW * NBPAD,), _i32),
                   pltpu.VMEM((NBPAD,), _i32),
                   pltpu.VMEM((16,), _i32),
                   pltpu.SemaphoreType.DMA],
)
def _sc_place(px, py, pz, hist, gx, gy, gz, ovc, pxl, pyl, pzl, histv, cur,
              ovv, sem):
    cid = lax.axis_index("c")
    sid = lax.axis_index("s")
    wid = cid * 16 + sid
    l = lax.iota(_i32, 16)
    base = pl.multiple_of(wid * PPT, PPT)
    pltpu.sync_copy(px.at[pl.ds(base, PPT)], pxl)
    pltpu.sync_copy(py.at[pl.ds(base, PPT)], pyl)
    pltpu.sync_copy(pz.at[pl.ds(base, PPT)], pzl)
    pltpu.sync_copy(hist, histv)

    widv = jnp.full((16,), wid, _i32)

    def cursor_chunk(c, carry):
        o = pl.multiple_of(c * 16, 16)
        acc = jnp.zeros((16,), _i32)
        for t in range(NW):
            h = histv[pl.ds(t * NBPAD + o, 16)]
            acc += jnp.where(jnp.full((16,), t, _i32) < widv, h, 0)
        cur[pl.ds(o, 16)] = acc
        return carry

    lax.fori_loop(0, NBPAD // 16, cursor_chunk, 0)

    ovbase = OVSTART + wid * PPT

    def chunk(k, ovcur):
        o = pl.multiple_of(k * 16, 16)
        b = _bin_ids(pxl[pl.ds(o, 16)], pyl[pl.ds(o, 16)])
        cnt, last = plsc.scan_count(b)
        c0 = plsc.load_gather(cur, [b])
        slot = c0 + cnt - 1
        plsc.store_scatter(cur, [b], c0 + cnt, mask=last)
        ov = slot >= CAP
        ovr = plsc.cumsum(jnp.where(ov, 1, 0).astype(_i32)) - 1
        dest = jnp.where(ov, ovbase + ovcur + ovr, b * CAP + slot)
        d1 = pltpu.async_copy(pxl.at[pl.ds(o, 16)], gx.at[dest], sem)
        d2 = pltpu.async_copy(pyl.at[pl.ds(o, 16)], gy.at[dest], sem)
        d3 = pltpu.async_copy(pzl.at[pl.ds(o, 16)], gz.at[dest], sem)
        d1.wait()
        d2.wait()
        d3.wait()
        return ovcur + jnp.sum(jnp.where(ov, 1, 0).astype(_i32))

    ovcur = lax.fori_loop(0, PPT // 16, chunk, jnp.int32(0))
    ovv[...] = jnp.full((16,), ovcur, _i32)
    obase = pl.multiple_of(wid * 16, 16)
    pltpu.sync_copy(ovv, ovc.at[pl.ds(obase, 16)])


@functools.partial(
    pl.kernel, mesh=_mesh, compiler_params=_sc_params,
    out_type=jax.ShapeDtypeStruct((A,), _f32),
    scratch_types=[pltpu.VMEM((NW * NBPAD,), _i32),
                   pltpu.VMEM((NBPAD,), _i32),
                   pltpu.VMEM((NW * 16,), _i32),
                   pltpu.VMEM((STRIPE,), _f32),
                   pltpu.VMEM((APT,), _f32),
                   pltpu.VMEM((APT,), _f32),
                   pltpu.VMEM((APT,), _i32),
                   pltpu.VMEM((APT,), _i32),
                   pltpu.VMEM((APT,), _i32),
                   pltpu.VMEM((APT,), _i32),
                   pltpu.VMEM((2 * CAP,), _f32),
                   pltpu.VMEM((2 * CAP,), _f32),
                   pltpu.VMEM((2 * CAP,), _f32),
                   pltpu.VMEM((16,), _f32),
                   pltpu.VMEM((16,), _f32),
                   pltpu.VMEM((APT,), _f32),
                   pltpu.VMEM_SHARED((NROWS,), _f32),
                   pltpu.VMEM_SHARED((NROWS,), _f32),
                   pltpu.VMEM_SHARED((NROWS,), _f32),
                   pltpu.SemaphoreType.DMA],
)
def _sc_score(gx, gy, gz, hist, ovc, lmx, lmy, scores, histv, tot, ovcv, stg,
              lmxl, lmyl, bx0a, bx1a, by0a, by1a, bufx, bufy, bufz,
              sacc, cacc, scl, shx, shy, shz, sem):
    cid = lax.axis_index("c")
    sid = lax.axis_index("s")
    wid = cid * 16 + sid
    l = lax.iota(_i32, 16)

    sbase = pl.multiple_of(sid * STRIPE, 8)
    pltpu.sync_copy(gx.at[pl.ds(sbase, STRIPE)], stg)
    pltpu.sync_copy(stg, shx.at[pl.ds(sbase, STRIPE)])
    pltpu.sync_copy(gy.at[pl.ds(sbase, STRIPE)], stg)
    pltpu.sync_copy(stg, shy.at[pl.ds(sbase, STRIPE)])
    pltpu.sync_copy(gz.at[pl.ds(sbase, STRIPE)], stg)
    pltpu.sync_copy(stg, shz.at[pl.ds(sbase, STRIPE)])

    pltpu.sync_copy(hist, histv)
    pltpu.sync_copy(ovc, ovcv)
    abase = pl.multiple_of(wid * APT, APT)
    pltpu.sync_copy(lmx.at[pl.ds(abase, APT)], lmxl)
    pltpu.sync_copy(lmy.at[pl.ds(abase, APT)], lmyl)

    def totchunk(c, carry):
        o = pl.multiple_of(c * 16, 16)
        acc = jnp.zeros((16,), _i32)
        for t in range(NW):
            acc += histv[pl.ds(t * NBPAD + o, 16)]
        tot[pl.ds(o, 16)] = acc
        return carry

    lax.fori_loop(0, NBPAD // 16, totchunk, 0)

    oacc = jnp.zeros((16,), _i32)
    for t in range(NW):
        oacc += ovcv[pl.ds(t * 16, 16)]
    ovtot = jnp.max(oacc)

    half = jnp.float32(ANCHOR_W / 2)
    for j in range(APT // 16):
        cxv = lmxl[pl.ds(j * 16, 16)]
        cyv = lmyl[pl.ds(j * 16, 16)]
        bx0a[pl.ds(j * 16, 16)] = jnp.clip(((cxv - half) * 0.25).astype(_i32), 0, G - 1)
        bx1a[pl.ds(j * 16, 16)] = jnp.clip(((cxv + half) * 0.25).astype(_i32), 0, G - 1)
        by0a[pl.ds(j * 16, 16)] = jnp.clip(((cyv - half) * 0.25).astype(_i32), 0, G - 1)
        by1a[pl.ds(j * 16, 16)] = jnp.clip(((cyv + half) * 0.25).astype(_i32), 0, G - 1)

    plsc.subcore_barrier()

    hw = jnp.float32(ANCHOR_W / 2)
    denom = hw * hw + jnp.float32(1e-6)

    def anchor_body(a, carry):
        cx = _gv(lmxl, a)
        cy = _gv(lmyl, a)
        bx0 = _gv(bx0a, a)
        bx1 = _gv(bx1a, a)
        by0 = _gv(by0a, a)
        by1 = _gv(by1a, a)
        sacc[...] = jnp.zeros((16,), _f32)
        cacc[...] = jnp.zeros((16,), _f32)
        cxv = jnp.full((16,), cx, _f32)
        cyv = jnp.full((16,), cy, _f32)

        def accum_chunk(xv, yv, zv, lane_ok):
            inbox = (lane_ok
                     & (xv >= cxv - half) & (xv <= cxv + half)
                     & (yv >= cyv - half) & (yv <= cyv + half)
                     & (zv >= 0.0) & (zv <= jnp.float32(ANCHOR_H)))
            dx = xv - cxv
            dy = yv - cyv
            r2 = dx * dx + dy * dy
            w = jnp.exp(-r2 / denom)
            sacc[...] += jnp.where(inbox, w, jnp.float32(0.0))
            cacc[...] += jnp.where(inbox, jnp.float32(1.0), jnp.float32(0.0))

        def bx_body(bx, carry2):
            b0 = bx * G + by0
            bbase = pl.multiple_of(b0 * CAP, CAP)
            pltpu.sync_copy(shx.at[pl.ds(bbase, 2 * CAP)], bufx)
            pltpu.sync_copy(shy.at[pl.ds(bbase, 2 * CAP)], bufy)
            pltpu.sync_copy(shz.at[pl.ds(bbase, 2 * CAP)], bufz)

            def by_body(by, carry3):
                b = bx * G + by
                n = jnp.minimum(_gv(tot, b), CAP)
                off = (by - by0) * CAP

                def chunk(k, carry4):
                    o = pl.multiple_of(off + k * 16, 16)
                    lane_ok = (l + k * 16) < n
                    accum_chunk(bufx[pl.ds(o, 16)], bufy[pl.ds(o, 16)],
                                bufz[pl.ds(o, 16)], lane_ok)
                    return carry4

                lax.fori_loop(0, (n + 15) // 16, chunk, 0)
                return carry3

            lax.fori_loop(by0, by1 + 1, by_body, 0)
            return carry2

        lax.fori_loop(bx0, bx1 + 1, bx_body, 0)

        @pl.when(ovtot > 0)
        def _():
            def t_body(t, carry2):
                ovt = _gv(ovcv, t * 16)

                def ovchunk(k, carry3):
                    obase = pl.multiple_of(OVSTART + t * PPT + k * 16, 16)
                    pltpu.sync_copy(shx.at[pl.ds(obase, 16)],
                                    bufx.at[pl.ds(0, 16)])
                    pltpu.sync_copy(shy.at[pl.ds(obase, 16)],
                                    bufy.at[pl.ds(0, 16)])
                    pltpu.sync_copy(shz.at[pl.ds(obase, 16)],
                                    bufz.at[pl.ds(0, 16)])
                    lane_ok = (l + k * 16) < ovt
                    accum_chunk(bufx[pl.ds(0, 16)], bufy[pl.ds(0, 16)],
                                bufz[pl.ds(0, 16)], lane_ok)
                    return carry3

                lax.fori_loop(0, (ovt + 15) // 16, ovchunk, 0)
                return carry2

            lax.fori_loop(0, NW, t_body, 0)

        s = jnp.sum(sacc[...])
        c = jnp.sum(cacc[...])
        val = jnp.full((16,), s, _f32) / (jnp.full((16,), c, _f32) + 1.0)
        plsc.store_scatter(scl, [jnp.full((16,), a, _i32)], val, mask=l == 0)
        return carry

    lax.fori_loop(0, APT, anchor_body, 0)
    pltpu.sync_copy(scl, scores.at[pl.ds(abase, APT)])


def _nms_body(score_c, score_r, cx_c, cy_c, cx_r, cy_r, supp_ref):
    s_i = score_c[...]
    s_j = score_r[...]

    half_w = jnp.float32(ANCHOR_W) / 2
    half_l = jnp.float32(ANCHOR_L) / 2
    x1_i = cx_c[...] - half_w
    y1_i = cy_c[...] - half_l
    x2_i = cx_c[...] + half_w
    y2_i = cy_c[...] + half_l
    x1_j = cx_r[...] - half_w
    y1_j = cy_r[...] - half_l
    x2_j = cx_r[...] + half_w
    y2_j = cy_r[...] + half_l
    area_i = (x2_i - x1_i) * (y2_i - y1_i)
    area_j = (x2_j - x1_j) * (y2_j - y1_j)
    ix1 = jnp.maximum(x1_i, x1_j)
    iy1 = jnp.maximum(y1_i, y1_j)
    ix2 = jnp.minimum(x2_i, x2_j)
    iy2 = jnp.minimum(y2_i, y2_j)
    iw = jnp.clip(ix2 - ix1, 0.0, None)
    ih = jnp.clip(iy2 - iy1, 0.0, None)
    inter = iw * ih
    union = area_i + area_j - inter
    iou = inter / (union + jnp.float32(1e-9))

    i_blk = pl.program_id(0)
    idx_i = i_blk * BA + jax.lax.broadcasted_iota(_i32, (BA, A), 0)
    idx_j = jax.lax.broadcasted_iota(_i32, (BA, A), 1)
    higher = (s_j > s_i) | ((s_j == s_i) & (idx_j < idx_i))
    suppressed = jnp.any(higher & (iou > jnp.float32(NMS_IOU)), axis=1,
                         keepdims=True)
    supp_ref[...] = suppressed.astype(_f32)


def _rank_body(score_c, supp_c, score_r, supp_r, rank_ref):
    neg_inf = jnp.float32(-jnp.inf)
    m_i = jnp.where(supp_c[...] > 0, neg_inf, score_c[...])
    m_j = jnp.where(supp_r[...] > 0, neg_inf, score_r[...])
    i_blk = pl.program_id(0)
    idx_i = i_blk * BA + jax.lax.broadcasted_iota(_i32, (BA, A), 0)
    idx_j = jax.lax.broadcasted_iota(_i32, (BA, A), 1)
    ahead = (m_j > m_i) | ((m_j == m_i) & (idx_j < idx_i))
    rank_ref[...] = ahead.astype(_i32).sum(axis=1, keepdims=True)


def _select_body(rank_r, score_r, supp_r, cx_r, cy_r, boxes_ref, top_ref):
    neg_inf = jnp.float32(-jnp.inf)
    m_j = jnp.where(supp_r[...] > 0, neg_inf, score_r[...])
    k = jax.lax.broadcasted_iota(_i32, (TOPK, A), 0)
    eq = rank_r[...] == k
    zero = jnp.float32(0.0)
    top_ref[...] = jnp.where(eq, m_j, zero).sum(axis=1, keepdims=True)
    bx = jnp.where(eq, cx_r[...], zero).sum(axis=1, keepdims=True)
    by = jnp.where(eq, cy_r[...], zero).sum(axis=1, keepdims=True)
    ones = jnp.ones((TOPK, 1), _f32)
    boxes_ref[...] = jnp.concatenate(
        [bx, by, jnp.zeros((TOPK, 1), _f32),
         ones * jnp.float32(ANCHOR_W), ones * jnp.float32(ANCHOR_L),
         ones * jnp.float32(ANCHOR_H)], axis=1)


def kernel(points, gt_boxes, local_maxima, plot_bounds, training):
    del gt_boxes, plot_bounds, training
    px = points[:, 0].astype(_f32)
    py = points[:, 1].astype(_f32)
    pz = points[:, 2].astype(_f32)
    lmx = local_maxima[:, 0].astype(_f32)
    lmy = local_maxima[:, 1].astype(_f32)

    hist = _sc_hist(px, py)
    gx, gy, gz, ovc = _sc_place(px, py, pz, hist)
    score_flat = _sc_score(gx, gy, gz, hist, ovc, lmx, lmy)

    score = score_flat.reshape(A, 1)
    score_r = score_flat.reshape(1, A)
    cx = lmx.reshape(A, 1)
    cy = lmy.reshape(A, 1)
    cx_r = lmx.reshape(1, A)
    cy_r = lmy.reshape(1, A)

    colb = pl.BlockSpec((BA, 1), lambda i: (i, 0))
    rowb = pl.BlockSpec((1, A), lambda i: (0, 0))
    supp = pl.pallas_call(
        _nms_body,
        grid=(A // BA,),
        in_specs=[colb, rowb, colb, colb, rowb, rowb],
        out_specs=colb,
        out_shape=jax.ShapeDtypeStruct((A, 1), _f32),
    )(score, score_r, cx, cy, cx_r, cy_r)

    supp_r = supp.reshape(1, A)
    rank = pl.pallas_call(
        _rank_body,
        grid=(A // BA,),
        in_specs=[colb, colb, rowb, rowb],
        out_specs=colb,
        out_shape=jax.ShapeDtypeStruct((A, 1), _i32),
    )(score, supp, score_r, supp_r)

    rank_r = rank.reshape(1, A)
    full = pl.BlockSpec((1, A), lambda: (0, 0))
    boxes, top = pl.pallas_call(
        _select_body,
        in_specs=[full, full, full, full, full],
        out_specs=[pl.BlockSpec((TOPK, 6), lambda: (0, 0)),
                   pl.BlockSpec((TOPK, 1), lambda: (0, 0))],
        out_shape=[jax.ShapeDtypeStruct((TOPK, 6), _f32),
                   jax.ShapeDtypeStruct((TOPK, 1), _f32)],
    )(rank_r, score_r, supp_r, cx_r, cy_r)

    return boxes, top.reshape(TOPK)


# speedup vs baseline: 2.5967x; 2.2386x over previous
import functools

import jax
import jax.numpy as jnp
from jax import lax
from jax.experimental import pallas as pl
from jax.experimental.pallas import tpu as pltpu, tpu_sc as plsc

P = 32768
A = 2048
ANCHOR_W = 4.0
ANCHOR_L = 4.0
ANCHOR_H = 15.0
NMS_IOU = 0.3
TOPK = 256

NT = 16
PPC = P // NT
APT = A // 32
G = 25
NB = G * G
CAP = 128
OVSTART = NB * CAP
NROWS = OVSTART + P
NBPAD = 640

BA = 256

_mesh = plsc.VectorSubcoreMesh(core_axis_name="c", subcore_axis_name="s")
_sc_params = pltpu.CompilerParams(needs_layout_passes=False)

_i32 = jnp.int32
_f32 = jnp.float32


def _bin_ids(xv, yv):
    bx = jnp.clip((xv * 0.25).astype(_i32), 0, G - 1)
    by = jnp.clip((yv * 0.25).astype(_i32), 0, G - 1)
    return bx * G + by


def _gv(ref, i):
    return jnp.max(plsc.load_gather(ref, [jnp.full((16,), i, _i32)]))


@functools.partial(
    pl.kernel, mesh=_mesh, compiler_params=_sc_params,
    out_type=jax.ShapeDtypeStruct((A,), _f32),
    scratch_types=[pltpu.VMEM((PPC,), _f32),
                   pltpu.VMEM((PPC,), _f32),
                   pltpu.VMEM((PPC,), _f32),
                   pltpu.VMEM((NBPAD,), _i32),
                   pltpu.VMEM((NT * NBPAD,), _i32),
                   pltpu.VMEM((NBPAD,), _i32),
                   pltpu.VMEM((NBPAD,), _i32),
                   pltpu.VMEM((NT * 16,), _i32),
                   pltpu.VMEM((16,), _i32),
                   pltpu.VMEM((APT,), _f32),
                   pltpu.VMEM((APT,), _f32),
                   pltpu.VMEM((APT,), _i32),
                   pltpu.VMEM((APT,), _i32),
                   pltpu.VMEM((APT,), _i32),
                   pltpu.VMEM((APT,), _i32),
                   pltpu.VMEM((2 * CAP,), _f32),
                   pltpu.VMEM((2 * CAP,), _f32),
                   pltpu.VMEM((2 * CAP,), _f32),
                   pltpu.VMEM((2 * CAP,), _f32),
                   pltpu.VMEM((2 * CAP,), _f32),
                   pltpu.VMEM((2 * CAP,), _f32),
                   pltpu.VMEM((16,), _f32),
                   pltpu.VMEM((16,), _f32),
                   pltpu.VMEM((APT,), _f32),
                   pltpu.VMEM_SHARED((NT * NBPAD,), _i32),
                   pltpu.VMEM_SHARED((NT * 16,), _i32),
                   pltpu.VMEM_SHARED((NROWS,), _f32),
                   pltpu.VMEM_SHARED((NROWS,), _f32),
                   pltpu.VMEM_SHARED((NROWS,), _f32),
                   pltpu.SemaphoreType.DMA],
)
def _sc_all(px, py, pz, lmx, lmy, scores, pxl, pyl, pzl, histl, histv, cur,
            tot, ovcv, ovv, lmxl, lmyl, bx0a, bx1a, by0a, by1a,
            bufx0, bufy0, bufz0, bufx1, bufy1, bufz1, sacc, cacc, scl,
            hist_sh, ovc_sh, shx, shy, shz, sem):
    cid = lax.axis_index("c")
    sid = lax.axis_index("s")
    wid = cid * NT + sid
    l = lax.iota(_i32, 16)

    base = pl.multiple_of(sid * PPC, PPC)
    pltpu.sync_copy(px.at[pl.ds(base, PPC)], pxl)
    pltpu.sync_copy(py.at[pl.ds(base, PPC)], pyl)
    pltpu.sync_copy(pz.at[pl.ds(base, PPC)], pzl)

    def zero(c, carry):
        histl[pl.ds(pl.multiple_of(c * 16, 16), 16)] = jnp.zeros((16,), _i32)
        return carry

    lax.fori_loop(0, NBPAD // 16, zero, 0)

    def hchunk(k, carry):
        o = pl.multiple_of(k * 16, 16)
        b = _bin_ids(pxl[pl.ds(o, 16)], pyl[pl.ds(o, 16)])
        cnt, last = plsc.scan_count(b)
        c0 = plsc.load_gather(histl, [b])
        plsc.store_scatter(histl, [b], c0 + cnt, mask=last)
        return carry

    lax.fori_loop(0, PPC // 16, hchunk, 0)
    hbase = pl.multiple_of(sid * NBPAD, NBPAD)
    pltpu.sync_copy(histl, hist_sh.at[pl.ds(hbase, NBPAD)])
    plsc.subcore_barrier()

    pltpu.sync_copy(hist_sh, histv)
    sidv = jnp.full((16,), sid, _i32)

    def cursor_chunk(c, carry):
        o = pl.multiple_of(c * 16, 16)
        acc = jnp.zeros((16,), _i32)
        mine = jnp.zeros((16,), _i32)
        for t in range(NT):
            h = histv[pl.ds(t * NBPAD + o, 16)]
            acc += h
            mine += jnp.where(jnp.full((16,), t, _i32) < sidv, h, 0)
        tot[pl.ds(o, 16)] = acc
        cur[pl.ds(o, 16)] = mine
        return carry

    lax.fori_loop(0, NBPAD // 16, cursor_chunk, 0)

    ovbase = OVSTART + sid * PPC

    def pchunk(k, ovcur):
        o = pl.multiple_of(k * 16, 16)
        b = _bin_ids(pxl[pl.ds(o, 16)], pyl[pl.ds(o, 16)])
        cnt, last = plsc.scan_count(b)
        c0 = plsc.load_gather(cur, [b])
        slot = c0 + cnt - 1
        plsc.store_scatter(cur, [b], c0 + cnt, mask=last)
        ov = slot >= CAP
        ovr = plsc.cumsum(jnp.where(ov, 1, 0).astype(_i32)) - 1
        dest = jnp.where(ov, ovbase + ovcur + ovr, b * CAP + slot)
        d1 = pltpu.async_copy(pxl.at[pl.ds(o, 16)], shx.at[dest], sem)
        d2 = pltpu.async_copy(pyl.at[pl.ds(o, 16)], shy.at[dest], sem)
        d3 = pltpu.async_copy(pzl.at[pl.ds(o, 16)], shz.at[dest], sem)
        d1.wait()
        d2.wait()
        d3.wait()
        return ovcur + jnp.sum(jnp.where(ov, 1, 0).astype(_i32))

    ovcur = lax.fori_loop(0, PPC // 16, pchunk, jnp.int32(0))
    ovv[...] = jnp.full((16,), ovcur, _i32)
    obase = pl.multiple_of(sid * 16, 16)
    pltpu.sync_copy(ovv, ovc_sh.at[pl.ds(obase, 16)])

    abase = pl.multiple_of(wid * APT, APT)
    pltpu.sync_copy(lmx.at[pl.ds(abase, APT)], lmxl)
    pltpu.sync_copy(lmy.at[pl.ds(abase, APT)], lmyl)
    half = jnp.float32(ANCHOR_W / 2)
    for j in range(APT // 16):
        cxv = lmxl[pl.ds(j * 16, 16)]
        cyv = lmyl[pl.ds(j * 16, 16)]
        bx0a[pl.ds(j * 16, 16)] = jnp.clip(((cxv - half) * 0.25).astype(_i32), 0, G - 1)
        bx1a[pl.ds(j * 16, 16)] = jnp.clip(((cxv + half) * 0.25).astype(_i32), 0, G - 1)
        by0a[pl.ds(j * 16, 16)] = jnp.clip(((cyv - half) * 0.25).astype(_i32), 0, G - 1)
        by1a[pl.ds(j * 16, 16)] = jnp.clip(((cyv + half) * 0.25).astype(_i32), 0, G - 1)

    plsc.subcore_barrier()

    pltpu.sync_copy(ovc_sh, ovcv)
    oacc = jnp.zeros((16,), _i32)
    for t in range(NT):
        oacc += ovcv[pl.ds(t * 16, 16)]
    ovtot = jnp.max(oacc)

    hw = jnp.float32(ANCHOR_W / 2)
    denom = hw * hw + jnp.float32(1e-6)

    def anchor_body(a, carry):
        cx = _gv(lmxl, a)
        cy = _gv(lmyl, a)
        bx0 = _gv(bx0a, a)
        bx1 = _gv(bx1a, a)
        by0 = _gv(by0a, a)
        by1 = _gv(by1a, a)
        sacc[...] = jnp.zeros((16,), _f32)
        cacc[...] = jnp.zeros((16,), _f32)
        cxv = jnp.full((16,), cx, _f32)
        cyv = jnp.full((16,), cy, _f32)

        b0 = bx0 * G + by0
        bb0 = pl.multiple_of(b0 * CAP, CAP)
        d0 = pltpu.async_copy(shx.at[pl.ds(bb0, 2 * CAP)], bufx0, sem)
        d1 = pltpu.async_copy(shy.at[pl.ds(bb0, 2 * CAP)], bufy0, sem)
        d2 = pltpu.async_copy(shz.at[pl.ds(bb0, 2 * CAP)], bufz0, sem)
        two_rows = bx1 > bx0

        @pl.when(two_rows)
        def _():
            b1 = bx1 * G + by0
            bb1 = pl.multiple_of(b1 * CAP, CAP)
            d3 = pltpu.async_copy(shx.at[pl.ds(bb1, 2 * CAP)], bufx1, sem)
            d4 = pltpu.async_copy(shy.at[pl.ds(bb1, 2 * CAP)], bufy1, sem)
            d5 = pltpu.async_copy(shz.at[pl.ds(bb1, 2 * CAP)], bufz1, sem)
            d3.wait()
            d4.wait()
            d5.wait()

        d0.wait()
        d1.wait()
        d2.wait()

        def accum_chunk(xv, yv, zv, lane_ok):
            inbox = (lane_ok
                     & (xv >= cxv - half) & (xv <= cxv + half)
                     & (yv >= cyv - half) & (yv <= cyv + half)
                     & (zv >= 0.0) & (zv <= jnp.float32(ANCHOR_H)))
            dx = xv - cxv
            dy = yv - cyv
            r2 = dx * dx + dy * dy
            w = jnp.exp(-r2 / denom)
            sacc[...] += jnp.where(inbox, w, jnp.float32(0.0))
            cacc[...] += jnp.where(inbox, jnp.float32(1.0), jnp.float32(0.0))

        def row_accum(bx, bufx, bufy, bufz):
            def by_body(by, carry3):
                b = bx * G + by
                n = jnp.minimum(_gv(tot, b), CAP)
                off = (by - by0) * CAP

                def chunk(k, carry4):
                    o = pl.multiple_of(off + k * 16, 16)
                    lane_ok = (l + k * 16) < n
                    accum_chunk(bufx[pl.ds(o, 16)], bufy[pl.ds(o, 16)],
                                bufz[pl.ds(o, 16)], lane_ok)
                    return carry4

                lax.fori_loop(0, (n + 15) // 16, chunk, 0)
                return carry3

            lax.fori_loop(by0, by1 + 1, by_body, 0)

        row_accum(bx0, bufx0, bufy0, bufz0)

        @pl.when(two_rows)
        def _():
            row_accum(bx1, bufx1, bufy1, bufz1)

        @pl.when(ovtot > 0)
        def _():
            def t_body(t, carry2):
                ovt = _gv(ovcv, t * 16)

                def ovchunk(k, carry3):
                    ob = pl.multiple_of(OVSTART + t * PPC + k * 16, 16)
                    pltpu.sync_copy(shx.at[pl.ds(ob, 16)],
                                    bufx0.at[pl.ds(0, 16)])
                    pltpu.sync_copy(shy.at[pl.ds(ob, 16)],
                                    bufy0.at[pl.ds(0, 16)])
                    pltpu.sync_copy(shz.at[pl.ds(ob, 16)],
                                    bufz0.at[pl.ds(0, 16)])
                    lane_ok = (l + k * 16) < ovt
                    accum_chunk(bufx0[pl.ds(0, 16)], bufy0[pl.ds(0, 16)],
                                bufz0[pl.ds(0, 16)], lane_ok)
                    return carry3

                lax.fori_loop(0, (ovt + 15) // 16, ovchunk, 0)
                return carry2

            lax.fori_loop(0, NT, t_body, 0)

        s = jnp.sum(sacc[...])
        c = jnp.sum(cacc[...])
        val = jnp.full((16,), s, _f32) / (jnp.full((16,), c, _f32) + 1.0)
        plsc.store_scatter(scl, [jnp.full((16,), a, _i32)], val, mask=l == 0)
        return carry

    lax.fori_loop(0, APT, anchor_body, 0)
    pltpu.sync_copy(scl, scores.at[pl.ds(abase, APT)])


def _nms_body(score_c, score_r, cx_c, cy_c, cx_r, cy_r, supp_ref):
    s_i = score_c[...]
    s_j = score_r[...]

    half_w = jnp.float32(ANCHOR_W) / 2
    half_l = jnp.float32(ANCHOR_L) / 2
    x1_i = cx_c[...] - half_w
    y1_i = cy_c[...] - half_l
    x2_i = cx_c[...] + half_w
    y2_i = cy_c[...] + half_l
    x1_j = cx_r[...] - half_w
    y1_j = cy_r[...] - half_l
    x2_j = cx_r[...] + half_w
    y2_j = cy_r[...] + half_l
    area_i = (x2_i - x1_i) * (y2_i - y1_i)
    area_j = (x2_j - x1_j) * (y2_j - y1_j)
    ix1 = jnp.maximum(x1_i, x1_j)
    iy1 = jnp.maximum(y1_i, y1_j)
    ix2 = jnp.minimum(x2_i, x2_j)
    iy2 = jnp.minimum(y2_i, y2_j)
    iw = jnp.clip(ix2 - ix1, 0.0, None)
    ih = jnp.clip(iy2 - iy1, 0.0, None)
    inter = iw * ih
    union = area_i + area_j - inter
    iou = inter / (union + jnp.float32(1e-9))

    i_blk = pl.program_id(0)
    idx_i = i_blk * BA + jax.lax.broadcasted_iota(_i32, (BA, A), 0)
    idx_j = jax.lax.broadcasted_iota(_i32, (BA, A), 1)
    higher = (s_j > s_i) | ((s_j == s_i) & (idx_j < idx_i))
    suppressed = jnp.any(higher & (iou > jnp.float32(NMS_IOU)), axis=1,
                         keepdims=True)
    supp_ref[...] = suppressed.astype(_f32)


def _rank_body(score_c, supp_c, score_r, supp_r, rank_ref):
    neg_inf = jnp.float32(-jnp.inf)
    m_i = jnp.where(supp_c[...] > 0, neg_inf, score_c[...])
    m_j = jnp.where(supp_r[...] > 0, neg_inf, score_r[...])
    i_blk = pl.program_id(0)
    idx_i = i_blk * BA + jax.lax.broadcasted_iota(_i32, (BA, A), 0)
    idx_j = jax.lax.broadcasted_iota(_i32, (BA, A), 1)
    ahead = (m_j > m_i) | ((m_j == m_i) & (idx_j < idx_i))
    rank_ref[...] = ahead.astype(_i32).sum(axis=1, keepdims=True)


def _select_body(rank_r, score_r, supp_r, cx_r, cy_r, boxes_ref, top_ref):
    neg_inf = jnp.float32(-jnp.inf)
    m_j = jnp.where(supp_r[...] > 0, neg_inf, score_r[...])
    k = jax.lax.broadcasted_iota(_i32, (TOPK, A), 0)
    eq = rank_r[...] == k
    zero = jnp.float32(0.0)
    top_ref[...] = jnp.where(eq, m_j, zero).sum(axis=1, keepdims=True)
    bx = jnp.where(eq, cx_r[...], zero).sum(axis=1, keepdims=True)
    by = jnp.where(eq, cy_r[...], zero).sum(axis=1, keepdims=True)
    ones = jnp.ones((TOPK, 1), _f32)
    boxes_ref[...] = jnp.concatenate(
        [bx, by, jnp.zeros((TOPK, 1), _f32),
         ones * jnp.float32(ANCHOR_W), ones * jnp.float32(ANCHOR_L),
         ones * jnp.float32(ANCHOR_H)], axis=1)


def kernel(points, gt_boxes, local_maxima, plot_bounds, training):
    del gt_boxes, plot_bounds, training
    px = points[:, 0].astype(_f32)
    py = points[:, 1].astype(_f32)
    pz = points[:, 2].astype(_f32)
    lmx = local_maxima[:, 0].astype(_f32)
    lmy = local_maxima[:, 1].astype(_f32)

    score_flat = _sc_all(px, py, pz, lmx, lmy)

    score = score_flat.reshape(A, 1)
    score_r = score_flat.reshape(1, A)
    cx = lmx.reshape(A, 1)
    cy = lmy.reshape(A, 1)
    cx_r = lmx.reshape(1, A)
    cy_r = lmy.reshape(1, A)

    colb = pl.BlockSpec((BA, 1), lambda i: (i, 0))
    rowb = pl.BlockSpec((1, A), lambda i: (0, 0))
    supp = pl.pallas_call(
        _nms_body,
        grid=(A // BA,),
        in_specs=[colb, rowb, colb, colb, rowb, rowb],
        out_specs=colb,
        out_shape=jax.ShapeDtypeStruct((A, 1), _f32),
    )(score, score_r, cx, cy, cx_r, cy_r)

    supp_r = supp.reshape(1, A)
    rank = pl.pallas_call(
        _rank_body,
        grid=(A // BA,),
        in_specs=[colb, colb, rowb, rowb],
        out_specs=colb,
        out_shape=jax.ShapeDtypeStruct((A, 1), _i32),
    )(score, supp, score_r, supp_r)

    rank_r = rank.reshape(1, A)
    full = pl.BlockSpec((1, A), lambda: (0, 0))
    boxes, top = pl.pallas_call(
        _select_body,
        in_specs=[full, full, full, full, full],
        out_specs=[pl.BlockSpec((TOPK, 6), lambda: (0, 0)),
                   pl.BlockSpec((TOPK, 1), lambda: (0, 0))],
        out_shape=[jax.ShapeDtypeStruct((TOPK, 6), _f32),
                   jax.ShapeDtypeStruct((TOPK, 1), _f32)],
    )(rank_r, score_r, supp_r, cx_r, cy_r)

    return boxes, top.reshape(TOPK)


# speedup vs baseline: 2.6382x; 1.0160x over previous
import functools

import jax
import jax.numpy as jnp
from jax import lax
from jax.experimental import pallas as pl
from jax.experimental.pallas import tpu as pltpu, tpu_sc as plsc

P = 32768
A = 2048
ANCHOR_W = 4.0
ANCHOR_L = 4.0
ANCHOR_H = 15.0
NMS_IOU = 0.3
TOPK = 256

NT = 16
PPC = P // NT
APT = A // 32
G = 25
NB = G * G
CAP = 128
OVSTART = NB * CAP
NROWS = OVSTART + P
NBPAD = 640

BA = 256

_mesh = plsc.VectorSubcoreMesh(core_axis_name="c", subcore_axis_name="s")
_sc_params = pltpu.CompilerParams(needs_layout_passes=False)

_i32 = jnp.int32
_f32 = jnp.float32


def _bin_ids(xv, yv):
    bx = jnp.clip((xv * 0.25).astype(_i32), 0, G - 1)
    by = jnp.clip((yv * 0.25).astype(_i32), 0, G - 1)
    return bx * G + by


def _gv(ref, i):
    return jnp.max(plsc.load_gather(ref, [jnp.full((16,), i, _i32)]))


@functools.partial(
    pl.kernel, mesh=_mesh, compiler_params=_sc_params,
    out_type=jax.ShapeDtypeStruct((A,), _f32),
    scratch_types=[pltpu.VMEM((PPC,), _f32),
                   pltpu.VMEM((PPC,), _f32),
                   pltpu.VMEM((PPC,), _f32),
                   pltpu.VMEM((NBPAD,), _i32),
                   pltpu.VMEM((NT * NBPAD,), _i32),
                   pltpu.VMEM((NBPAD,), _i32),
                   pltpu.VMEM((NBPAD,), _i32),
                   pltpu.VMEM((NT * 16,), _i32),
                   pltpu.VMEM((16,), _i32),
                   pltpu.VMEM((APT,), _f32),
                   pltpu.VMEM((APT,), _f32),
                   pltpu.VMEM((APT,), _i32),
                   pltpu.VMEM((APT,), _i32),
                   pltpu.VMEM((APT,), _i32),
                   pltpu.VMEM((APT,), _i32),
                   pltpu.VMEM((2 * CAP,), _f32),
                   pltpu.VMEM((2 * CAP,), _f32),
                   pltpu.VMEM((2 * CAP,), _f32),
                   pltpu.VMEM((2 * CAP,), _f32),
                   pltpu.VMEM((16,), _f32),
                   pltpu.VMEM((16,), _f32),
                   pltpu.VMEM((APT,), _f32),
                   pltpu.VMEM_SHARED((NT * NBPAD,), _i32),
                   pltpu.VMEM_SHARED((NT * 16,), _i32),
                   pltpu.VMEM_SHARED((NROWS,), _f32),
                   pltpu.VMEM_SHARED((NROWS,), _f32),
                   pltpu.SemaphoreType.DMA],
)
def _sc_all(px, py, pz, lmx, lmy, scores, pxl, pyl, pzl, histl, histv, cur,
            tot, ovcv, ovv, lmxl, lmyl, bx0a, bx1a, by0a, by1a,
            bufx0, bufy0, bufx1, bufy1, sacc, cacc, scl,
            hist_sh, ovc_sh, shx, shy, sem):
    cid = lax.axis_index("c")
    sid = lax.axis_index("s")
    wid = cid * NT + sid
    l = lax.iota(_i32, 16)

    base = pl.multiple_of(sid * PPC, PPC)
    pltpu.sync_copy(px.at[pl.ds(base, PPC)], pxl)
    pltpu.sync_copy(py.at[pl.ds(base, PPC)], pyl)
    pltpu.sync_copy(pz.at[pl.ds(base, PPC)], pzl)

    def zero(c, carry):
        histl[pl.ds(pl.multiple_of(c * 16, 16), 16)] = jnp.zeros((16,), _i32)
        return carry

    lax.fori_loop(0, NBPAD // 16, zero, 0)

    def hchunk(k, carry):
        o = pl.multiple_of(k * 16, 16)
        b = _bin_ids(pxl[pl.ds(o, 16)], pyl[pl.ds(o, 16)])
        cnt, last = plsc.scan_count(b)
        c0 = plsc.load_gather(histl, [b])
        plsc.store_scatter(histl, [b], c0 + cnt, mask=last)
        return carry

    lax.fori_loop(0, PPC // 16, hchunk, 0)
    hbase = pl.multiple_of(sid * NBPAD, NBPAD)
    pltpu.sync_copy(histl, hist_sh.at[pl.ds(hbase, NBPAD)])
    plsc.subcore_barrier()

    pltpu.sync_copy(hist_sh, histv)
    sidv = jnp.full((16,), sid, _i32)

    def cursor_chunk(c, carry):
        o = pl.multiple_of(c * 16, 16)
        acc = jnp.zeros((16,), _i32)
        mine = jnp.zeros((16,), _i32)
        for t in range(NT):
            h = histv[pl.ds(t * NBPAD + o, 16)]
            acc += h
            mine += jnp.where(jnp.full((16,), t, _i32) < sidv, h, 0)
        tot[pl.ds(o, 16)] = acc
        cur[pl.ds(o, 16)] = mine
        return carry

    lax.fori_loop(0, NBPAD // 16, cursor_chunk, 0)

    ovbase = OVSTART + sid * PPC

    def pchunk(k, ovcur):
        o = pl.multiple_of(k * 16, 16)
        xv = pxl[pl.ds(o, 16)]
        b = _bin_ids(xv, pyl[pl.ds(o, 16)])
        cnt, last = plsc.scan_count(b)
        c0 = plsc.load_gather(cur, [b])
        slot = c0 + cnt - 1
        plsc.store_scatter(cur, [b], c0 + cnt, mask=last)
        ov = slot >= CAP
        ovr = plsc.cumsum(jnp.where(ov, 1, 0).astype(_i32)) - 1
        dest = jnp.where(ov, ovbase + ovcur + ovr, b * CAP + slot)
        zv = pzl[pl.ds(o, 16)]
        zok = (zv >= 0.0) & (zv <= jnp.float32(ANCHOR_H))
        pzl[pl.ds(o, 16)] = jnp.where(zok, xv, jnp.float32(1e9))
        d1 = pltpu.async_copy(pzl.at[pl.ds(o, 16)], shx.at[dest], sem)
        d2 = pltpu.async_copy(pyl.at[pl.ds(o, 16)], shy.at[dest], sem)
        d1.wait()
        d2.wait()
        return ovcur + jnp.sum(jnp.where(ov, 1, 0).astype(_i32))

    ovcur = lax.fori_loop(0, PPC // 16, pchunk, jnp.int32(0))
    ovv[...] = jnp.full((16,), ovcur, _i32)
    obase = pl.multiple_of(sid * 16, 16)
    pltpu.sync_copy(ovv, ovc_sh.at[pl.ds(obase, 16)])

    abase = pl.multiple_of(wid * APT, APT)
    pltpu.sync_copy(lmx.at[pl.ds(abase, APT)], lmxl)
    pltpu.sync_copy(lmy.at[pl.ds(abase, APT)], lmyl)
    half = jnp.float32(ANCHOR_W / 2)
    for j in range(APT // 16):
        cxv = lmxl[pl.ds(j * 16, 16)]
        cyv = lmyl[pl.ds(j * 16, 16)]
        bx0a[pl.ds(j * 16, 16)] = jnp.clip(((cxv - half) * 0.25).astype(_i32), 0, G - 1)
        bx1a[pl.ds(j * 16, 16)] = jnp.clip(((cxv + half) * 0.25).astype(_i32), 0, G - 1)
        by0a[pl.ds(j * 16, 16)] = jnp.clip(((cyv - half) * 0.25).astype(_i32), 0, G - 1)
        by1a[pl.ds(j * 16, 16)] = jnp.clip(((cyv + half) * 0.25).astype(_i32), 0, G - 1)

    plsc.subcore_barrier()

    pltpu.sync_copy(ovc_sh, ovcv)
    oacc = jnp.zeros((16,), _i32)
    for t in range(NT):
        oacc += ovcv[pl.ds(t * 16, 16)]
    ovtot = jnp.max(oacc)

    hw = jnp.float32(ANCHOR_W / 2)
    denom = hw * hw + jnp.float32(1e-6)

    def anchor_body(a, carry):
        cx = _gv(lmxl, a)
        cy = _gv(lmyl, a)
        bx0 = _gv(bx0a, a)
        bx1 = _gv(bx1a, a)
        by0 = _gv(by0a, a)
        by1 = _gv(by1a, a)
        sacc[...] = jnp.zeros((16,), _f32)
        cacc[...] = jnp.zeros((16,), _f32)
        cxv = jnp.full((16,), cx, _f32)
        cyv = jnp.full((16,), cy, _f32)

        b0 = bx0 * G + by0
        bb0 = pl.multiple_of(b0 * CAP, CAP)
        d0 = pltpu.async_copy(shx.at[pl.ds(bb0, 2 * CAP)], bufx0, sem)
        d1 = pltpu.async_copy(shy.at[pl.ds(bb0, 2 * CAP)], bufy0, sem)
        two_rows = bx1 > bx0

        @pl.when(two_rows)
        def _():
            b1 = bx1 * G + by0
            bb1 = pl.multiple_of(b1 * CAP, CAP)
            d3 = pltpu.async_copy(shx.at[pl.ds(bb1, 2 * CAP)], bufx1, sem)
            d4 = pltpu.async_copy(shy.at[pl.ds(bb1, 2 * CAP)], bufy1, sem)
            d3.wait()
            d4.wait()

        d0.wait()
        d1.wait()

        def accum_chunk(xv, yv, lane_ok):
            inbox = (lane_ok
                     & (xv >= cxv - half) & (xv <= cxv + half)
                     & (yv >= cyv - half) & (yv <= cyv + half))
            dx = xv - cxv
            dy = yv - cyv
            r2 = dx * dx + dy * dy
            w = jnp.exp(-r2 / denom)
            sacc[...] += jnp.where(inbox, w, jnp.float32(0.0))
            cacc[...] += jnp.where(inbox, jnp.float32(1.0), jnp.float32(0.0))

        def row_accum(bx, bufx, bufy):
            def by_body(by, carry3):
                b = bx * G + by
                n = jnp.minimum(_gv(tot, b), CAP)
                off = (by - by0) * CAP

                def chunk(k, carry4):
                    o = pl.multiple_of(off + k * 16, 16)
                    lane_ok = (l + k * 16) < n
                    accum_chunk(bufx[pl.ds(o, 16)], bufy[pl.ds(o, 16)],
                                lane_ok)
                    return carry4

                lax.fori_loop(0, (n + 15) // 16, chunk, 0)
                return carry3

            lax.fori_loop(by0, by1 + 1, by_body, 0)

        row_accum(bx0, bufx0, bufy0)

        @pl.when(two_rows)
        def _():
            row_accum(bx1, bufx1, bufy1)

        @pl.when(ovtot > 0)
        def _():
            def t_body(t, carry2):
                ovt = _gv(ovcv, t * 16)

                def ovchunk(k, carry3):
                    ob = pl.multiple_of(OVSTART + t * PPC + k * 16, 16)
                    pltpu.sync_copy(shx.at[pl.ds(ob, 16)],
                                    bufx0.at[pl.ds(0, 16)])
                    pltpu.sync_copy(shy.at[pl.ds(ob, 16)],
                                    bufy0.at[pl.ds(0, 16)])
                    lane_ok = (l + k * 16) < ovt
                    accum_chunk(bufx0[pl.ds(0, 16)], bufy0[pl.ds(0, 16)],
                                lane_ok)
                    return carry3

                lax.fori_loop(0, (ovt + 15) // 16, ovchunk, 0)
                return carry2

            lax.fori_loop(0, NT, t_body, 0)

        s = jnp.sum(sacc[...])
        c = jnp.sum(cacc[...])
        val = jnp.full((16,), s, _f32) / (jnp.full((16,), c, _f32) + 1.0)
        plsc.store_scatter(scl, [jnp.full((16,), a, _i32)], val, mask=l == 0)
        return carry

    lax.fori_loop(0, APT, anchor_body, 0)
    pltpu.sync_copy(scl, scores.at[pl.ds(abase, APT)])


def _nms_body(score_c, score_r, cx_c, cy_c, cx_r, cy_r, supp_ref):
    s_i = score_c[...]
    s_j = score_r[...]

    half_w = jnp.float32(ANCHOR_W) / 2
    half_l = jnp.float32(ANCHOR_L) / 2
    x1_i = cx_c[...] - half_w
    y1_i = cy_c[...] - half_l
    x2_i = cx_c[...] + half_w
    y2_i = cy_c[...] + half_l
    x1_j = cx_r[...] - half_w
    y1_j = cy_r[...] - half_l
    x2_j = cx_r[...] + half_w
    y2_j = cy_r[...] + half_l
    area_i = (x2_i - x1_i) * (y2_i - y1_i)
    area_j = (x2_j - x1_j) * (y2_j - y1_j)
    ix1 = jnp.maximum(x1_i, x1_j)
    iy1 = jnp.maximum(y1_i, y1_j)
    ix2 = jnp.minimum(x2_i, x2_j)
    iy2 = jnp.minimum(y2_i, y2_j)
    iw = jnp.clip(ix2 - ix1, 0.0, None)
    ih = jnp.clip(iy2 - iy1, 0.0, None)
    inter = iw * ih
    union = area_i + area_j - inter
    iou = inter / (union + jnp.float32(1e-9))

    i_blk = pl.program_id(0)
    idx_i = i_blk * BA + jax.lax.broadcasted_iota(_i32, (BA, A), 0)
    idx_j = jax.lax.broadcasted_iota(_i32, (BA, A), 1)
    higher = (s_j > s_i) | ((s_j == s_i) & (idx_j < idx_i))
    suppressed = jnp.any(higher & (iou > jnp.float32(NMS_IOU)), axis=1,
                         keepdims=True)
    supp_ref[...] = suppressed.astype(_f32)


def _rank_body(score_c, supp_c, score_r, supp_r, rank_ref):
    neg_inf = jnp.float32(-jnp.inf)
    m_i = jnp.where(supp_c[...] > 0, neg_inf, score_c[...])
    m_j = jnp.where(supp_r[...] > 0, neg_inf, score_r[...])
    i_blk = pl.program_id(0)
    idx_i = i_blk * BA + jax.lax.broadcasted_iota(_i32, (BA, A), 0)
    idx_j = jax.lax.broadcasted_iota(_i32, (BA, A), 1)
    ahead = (m_j > m_i) | ((m_j == m_i) & (idx_j < idx_i))
    rank_ref[...] = ahead.astype(_i32).sum(axis=1, keepdims=True)


def _select_body(rank_r, score_r, supp_r, cx_r, cy_r, boxes_ref, top_ref):
    neg_inf = jnp.float32(-jnp.inf)
    m_j = jnp.where(supp_r[...] > 0, neg_inf, score_r[...])
    k = jax.lax.broadcasted_iota(_i32, (TOPK, A), 0)
    eq = rank_r[...] == k
    zero = jnp.float32(0.0)
    top_ref[...] = jnp.where(eq, m_j, zero).sum(axis=1, keepdims=True)
    bx = jnp.where(eq, cx_r[...], zero).sum(axis=1, keepdims=True)
    by = jnp.where(eq, cy_r[...], zero).sum(axis=1, keepdims=True)
    ones = jnp.ones((TOPK, 1), _f32)
    boxes_ref[...] = jnp.concatenate(
        [bx, by, jnp.zeros((TOPK, 1), _f32),
         ones * jnp.float32(ANCHOR_W), ones * jnp.float32(ANCHOR_L),
         ones * jnp.float32(ANCHOR_H)], axis=1)


def kernel(points, gt_boxes, local_maxima, plot_bounds, training):
    del gt_boxes, plot_bounds, training
    px = points[:, 0].astype(_f32)
    py = points[:, 1].astype(_f32)
    pz = points[:, 2].astype(_f32)
    lmx = local_maxima[:, 0].astype(_f32)
    lmy = local_maxima[:, 1].astype(_f32)

    score_flat = _sc_all(px, py, pz, lmx, lmy)

    score = score_flat.reshape(A, 1)
    score_r = score_flat.reshape(1, A)
    cx = lmx.reshape(A, 1)
    cy = lmy.reshape(A, 1)
    cx_r = lmx.reshape(1, A)
    cy_r = lmy.reshape(1, A)

    colb = pl.BlockSpec((BA, 1), lambda i: (i, 0))
    rowb = pl.BlockSpec((1, A), lambda i: (0, 0))
    supp = pl.pallas_call(
        _nms_body,
        grid=(A // BA,),
        in_specs=[colb, rowb, colb, colb, rowb, rowb],
        out_specs=colb,
        out_shape=jax.ShapeDtypeStruct((A, 1), _f32),
    )(score, score_r, cx, cy, cx_r, cy_r)

    supp_r = supp.reshape(1, A)
    rank = pl.pallas_call(
        _rank_body,
        grid=(A // BA,),
        in_specs=[colb, colb, rowb, rowb],
        out_specs=colb,
        out_shape=jax.ShapeDtypeStruct((A, 1), _i32),
    )(score, supp, score_r, supp_r)

    rank_r = rank.reshape(1, A)
    full = pl.BlockSpec((1, A), lambda: (0, 0))
    boxes, top = pl.pallas_call(
        _select_body,
        in_specs=[full, full, full, full, full],
        out_specs=[pl.BlockSpec((TOPK, 6), lambda: (0, 0)),
                   pl.BlockSpec((TOPK, 1), lambda: (0, 0))],
        out_shape=[jax.ShapeDtypeStruct((TOPK, 6), _f32),
                   jax.ShapeDtypeStruct((TOPK, 1), _f32)],
    )(rank_r, score_r, supp_r, cx_r, cy_r)

    return boxes, top.reshape(TOPK)
